# baseline (speedup 1.0000x reference)
import jax
import jax.numpy as jnp
from jax import lax
from jax.experimental import pallas as pl
from jax.experimental.pallas import tpu as pltpu
from jax.experimental.pallas import tpu_sc as plsc

B = 16384
D = 64
L = 16
NW = 32
BPW = B // NW
NCHUNK = 4
CHUNK = BPW // NCHUNK


def _sc_body(u_hbm, i_hbm, j_hbm, uw_hbm, iw_hbm, xp_out, sq_out,
             idx_u, idx_i, idx_j, u_rows, i_rows, j_rows, xp_v, sq_v,
             sem):
    wid = lax.axis_index("s") * 2 + lax.axis_index("c")
    base = wid * BPW

    pltpu.sync_copy(u_hbm.at[pl.ds(base, BPW)], idx_u)
    pltpu.sync_copy(i_hbm.at[pl.ds(base, BPW)], idx_i)
    pltpu.sync_copy(j_hbm.at[pl.ds(base, BPW)], idx_j)

    descs = []
    for c in range(NCHUNK):
        sl = pl.ds(c * CHUNK, CHUNK)
        descs.append(pltpu.async_copy(uw_hbm.at[idx_u.at[sl]], u_rows.at[sl], sem))
        descs.append(pltpu.async_copy(iw_hbm.at[idx_i.at[sl]], i_rows.at[sl], sem))
        descs.append(pltpu.async_copy(iw_hbm.at[idx_j.at[sl]], j_rows.at[sl], sem))
    for dsc in descs:
        dsc.wait()

    zero = jnp.zeros((L,), jnp.float32)

    def group(g, acc):
        accs = list(acc)
        for rr in range(L):
            r = g * L + rr
            us = [u_rows[r, pl.ds(k * L, L)] for k in range(4)]
            is_ = [i_rows[r, pl.ds(k * L, L)] for k in range(4)]
            js = [j_rows[r, pl.ds(k * L, L)] for k in range(4)]
            t = [us[k] * (is_[k] - js[k]) for k in range(4)]
            p = (t[0] + t[1]) + (t[2] + t[3])
            for k in range(4):
                accs[k] = accs[k] + us[k] * us[k]
                accs[4 + k] = accs[4 + k] + is_[k] * is_[k]
                accs[8 + k] = accs[8 + k] + js[k] * js[k]
            xp_v[pl.ds(r * L, L)] = p
        return tuple(accs)

    accs = lax.fori_loop(0, BPW // L, group, tuple([zero] * 12))
    sq = accs[0]
    for k in range(1, 12):
        sq = sq + accs[k]
    sq_v[...] = sq

    pltpu.sync_copy(xp_v, xp_out.at[pl.ds(base * L, BPW * L)])
    pltpu.sync_copy(sq_v, sq_out.at[wid])


def _make_sc_kernel():
    mesh = plsc.VectorSubcoreMesh(core_axis_name="c", subcore_axis_name="s")
    return pl.kernel(
        _sc_body,
        mesh=mesh,
        compiler_params=pltpu.CompilerParams(use_tc_tiling_on_sc=False),
        out_type=[
            jax.ShapeDtypeStruct((B * L,), jnp.float32),
            jax.ShapeDtypeStruct((NW, L), jnp.float32),
        ],
        scratch_types=[
            pltpu.VMEM((BPW,), jnp.int32),
            pltpu.VMEM((BPW,), jnp.int32),
            pltpu.VMEM((BPW,), jnp.int32),
            pltpu.VMEM((BPW, D), jnp.float32),
            pltpu.VMEM((BPW, D), jnp.float32),
            pltpu.VMEM((BPW, D), jnp.float32),
            pltpu.VMEM((BPW * L,), jnp.float32),
            pltpu.VMEM((L,), jnp.float32),
            pltpu.SemaphoreType.DMA,
        ],
    )


def _finish_body(xp_ref, sq_ref, auc_ref, loss_ref, l2_ref):
    xp = xp_ref[...]
    sq = sq_ref[...]
    colg = lax.broadcasted_iota(jnp.int32, (128, 8), 0) // L
    outg = lax.broadcasted_iota(jnp.int32, (128, 8), 1)
    m = (colg == outg).astype(jnp.float32)
    x = jnp.dot(xp, m, preferred_element_type=jnp.float32)
    inv_b = 1.0 / B
    auc_ref[0, 0] = jnp.sum((x > 0.0).astype(jnp.float32)) * inv_b
    l2 = jnp.sum(sq)
    l2_ref[0, 0] = l2
    ls = jnp.minimum(x, 0.0) - jnp.log1p(jnp.exp(-jnp.abs(x)))
    loss_ref[0, 0] = 1e-4 * l2 - jnp.sum(ls) * inv_b


def kernel(u, i, j, user_emb_w, item_emb_w):
    sc = _make_sc_kernel()
    xp, sq = sc(u.astype(jnp.int32), i.astype(jnp.int32), j.astype(jnp.int32),
                user_emb_w, item_emb_w)
    xp2d = xp.reshape(B * L // 128, 128)
    sq2d = sq.reshape(4, 128)
    auc, loss, l2 = pl.pallas_call(
        _finish_body,
        out_shape=[jax.ShapeDtypeStruct((1, 1), jnp.float32)] * 3,
        out_specs=[pl.BlockSpec(memory_space=pltpu.SMEM)] * 3,
    )(xp2d, sq2d)
    return (auc.reshape(()), loss.reshape(()), l2.reshape(()))


# speedup vs baseline: 1.4009x; 1.4009x over previous
import jax
import jax.numpy as jnp
from jax import lax
from jax.experimental import pallas as pl
from jax.experimental.pallas import tpu as pltpu
from jax.experimental.pallas import tpu_sc as plsc

B = 16384
D = 64
L = 16
NW = 32
BPW = B // NW
CR = 32
NCH = BPW // CR


def _sc_body(u_hbm, i_hbm, j_hbm, uw_hbm, iw_hbm, xp_out, sq_out,
             idx_u, idx_i, idx_j, ub, ib, jb, xp_v, sq_v, sem):
    wid = lax.axis_index("s") * 2 + lax.axis_index("c")
    base = wid * BPW

    pltpu.sync_copy(u_hbm.at[pl.ds(base, BPW)], idx_u.at[pl.ds(0, BPW)])
    pltpu.sync_copy(i_hbm.at[pl.ds(base, BPW)], idx_i.at[pl.ds(0, BPW)])
    pltpu.sync_copy(j_hbm.at[pl.ds(base, BPW)], idx_j.at[pl.ds(0, BPW)])

    tables = ((idx_u, uw_hbm, ub), (idx_i, iw_hbm, ib), (idx_j, iw_hbm, jb))
    zero = jnp.zeros((L,), jnp.float32)

    def chunk(c, acc):
        accs = list(acc)
        r0 = c * CR
        for rr in range(CR):
            for idxb, tbl, buf in tables:
                e = idxb[pl.ds(r0 + rr, L)][0]
                rb = pl.multiple_of((e >> 3) << 3, 8)
                pltpu.async_copy(tbl.at[pl.ds(rb, 8), :],
                                 buf.at[pl.ds(rr * 8, 8), :], sem)
        for idxb, tbl, buf in tables:
            pltpu.make_async_copy(tbl.at[pl.ds(0, CR * 8), :], buf, sem).wait()
        for rr in range(CR):
            r = r0 + rr
            eu = idx_u[pl.ds(r, L)][0]
            ei = idx_i[pl.ds(r, L)][0]
            ej = idx_j[pl.ds(r, L)][0]
            ru = rr * 8 + (eu & 7)
            ri = rr * 8 + (ei & 7)
            rj = rr * 8 + (ej & 7)
            us = [ub[ru, pl.ds(k * L, L)] for k in range(4)]
            is_ = [ib[ri, pl.ds(k * L, L)] for k in range(4)]
            js = [jb[rj, pl.ds(k * L, L)] for k in range(4)]
            t = [us[k] * (is_[k] - js[k]) for k in range(4)]
            p = (t[0] + t[1]) + (t[2] + t[3])
            for k in range(4):
                accs[k] = accs[k] + us[k] * us[k]
                accs[4 + k] = accs[4 + k] + is_[k] * is_[k]
                accs[8 + k] = accs[8 + k] + js[k] * js[k]
            xp_v[pl.ds(r * L, L)] = p
        return tuple(accs)

    accs = lax.fori_loop(0, NCH, chunk, tuple([zero] * 12))
    sq = accs[0]
    for k in range(1, 12):
        sq = sq + accs[k]
    sq_v[...] = sq

    pltpu.sync_copy(xp_v, xp_out.at[pl.ds(base * L, BPW * L)])
    pltpu.sync_copy(sq_v, sq_out.at[wid])


def _make_sc_kernel():
    mesh = plsc.VectorSubcoreMesh(core_axis_name="c", subcore_axis_name="s")
    return pl.kernel(
        _sc_body,
        mesh=mesh,
        compiler_params=pltpu.CompilerParams(use_tc_tiling_on_sc=True),
        out_type=[
            jax.ShapeDtypeStruct((B * L,), jnp.float32),
            jax.ShapeDtypeStruct((NW, L), jnp.float32),
        ],
        scratch_types=[
            pltpu.VMEM((BPW + L,), jnp.int32),
            pltpu.VMEM((BPW + L,), jnp.int32),
            pltpu.VMEM((BPW + L,), jnp.int32),
            pltpu.VMEM((CR * 8, D), jnp.float32),
            pltpu.VMEM((CR * 8, D), jnp.float32),
            pltpu.VMEM((CR * 8, D), jnp.float32),
            pltpu.VMEM((BPW * L,), jnp.float32),
            pltpu.VMEM((L,), jnp.float32),
            pltpu.SemaphoreType.DMA,
        ],
    )


def _finish_body(xp_ref, sq_ref, auc_ref, loss_ref, l2_ref):
    xp = xp_ref[...]
    sq = sq_ref[...]
    colg = lax.broadcasted_iota(jnp.int32, (128, 8), 0) // L
    outg = lax.broadcasted_iota(jnp.int32, (128, 8), 1)
    m = (colg == outg).astype(jnp.float32)
    x = jnp.dot(xp, m, preferred_element_type=jnp.float32)
    inv_b = 1.0 / B
    auc_ref[0, 0] = jnp.sum((x > 0.0).astype(jnp.float32)) * inv_b
    l2 = jnp.sum(sq)
    l2_ref[0, 0] = l2
    ls = jnp.minimum(x, 0.0) - jnp.log1p(jnp.exp(-jnp.abs(x)))
    loss_ref[0, 0] = 1e-4 * l2 - jnp.sum(ls) * inv_b


def kernel(u, i, j, user_emb_w, item_emb_w):
    sc = _make_sc_kernel()
    xp, sq = sc(u.astype(jnp.int32), i.astype(jnp.int32), j.astype(jnp.int32),
                user_emb_w, item_emb_w)
    xp2d = xp.reshape(B * L // 128, 128)
    sq2d = sq.reshape(4, 128)
    auc, loss, l2 = pl.pallas_call(
        _finish_body,
        out_shape=[jax.ShapeDtypeStruct((1, 1), jnp.float32)] * 3,
        out_specs=[pl.BlockSpec(memory_space=pltpu.SMEM)] * 3,
    )(xp2d, sq2d)
    return (auc.reshape(()), loss.reshape(()), l2.reshape(()))
